# baseline (speedup 1.0000x reference)
import functools

import jax
import jax.numpy as jnp
from jax import lax
from jax.experimental import pallas as pl
from jax.experimental.pallas import tpu as pltpu
from jax.experimental.pallas import tpu_sc as plsc

B, N, D, K = 64, 1024, 128, 256
NUM_SQUARINGS = 7


def _assign_kernel(cent_ref, c2_ref, descs_ref, clusters_ref):
    cent = cent_ref[...]
    c2 = c2_ref[...]
    d = descs_ref[0]
    scores = jax.lax.dot_general(
        d, cent, (((1,), (1,)), ((), ())),
        preferred_element_type=jnp.float32,
        precision=jax.lax.Precision.HIGHEST)
    dist = c2 - 2.0 * scores
    clusters_ref[0, 0] = jnp.argmin(dist, axis=-1)


def _tc_assign(centroids, c2, descs):
    clusters = pl.pallas_call(
        _assign_kernel,
        grid=(B,),
        in_specs=[
            pl.BlockSpec((K, D), lambda b: (0, 0)),
            pl.BlockSpec((1, K), lambda b: (0, 0)),
            pl.BlockSpec((1, N, D), lambda b: (b, 0, 0)),
        ],
        out_specs=pl.BlockSpec((1, 1, N), lambda b: (b, 0, 0)),
        out_shape=jax.ShapeDtypeStruct((B, 1, N), jnp.int32),
    )(centroids, c2, descs)
    return clusters.reshape(B, N)


_SC_INFO = plsc.get_sparse_core_info()
_NC = _SC_INFO.num_cores
_NS = _SC_INFO.num_subcores
_IMGS_PER_CORE = B // _NC
_ROWS_PER_TILE = N // _NS
_OUT_ROWS = K // _NS
_PW = 128


def _sc_segment_sum(descs, clusters):
    mesh = plsc.VectorSubcoreMesh(core_axis_name="c", subcore_axis_name="s")

    @functools.partial(
        pl.kernel,
        out_type=(jax.ShapeDtypeStruct((B, K, D), jnp.float32),
                  jax.ShapeDtypeStruct((B, K, _PW), jnp.float32)),
        mesh=mesh,
        scratch_types=[
            pltpu.VMEM((_ROWS_PER_TILE, D), jnp.float32),
            pltpu.VMEM((_ROWS_PER_TILE,), jnp.int32),
            pltpu.VMEM((_ROWS_PER_TILE, _PW), jnp.float32),
            pltpu.VMEM((_OUT_ROWS, D), jnp.float32),
            pltpu.VMEM_SHARED((K, D), jnp.float32),
            pltpu.VMEM_SHARED((K, _PW), jnp.float32),
            pltpu.SemaphoreType.DMA,
        ],
    )
    def seg_sum(descs_hbm, clusters_hbm, out_hbm, pops_hbm, rows_v, idx_v,
                ones_v, zero_v, acc_sh, pacc_sh, sem):
        cid = lax.axis_index("c")
        sid = lax.axis_index("s")
        zeros16 = jnp.zeros((16,), jnp.float32)
        one_hot16 = jnp.where(lax.iota(jnp.int32, 16) == 0, 1.0, 0.0
                              ).astype(jnp.float32)
        for i in range(_OUT_ROWS):
            for j in range(D // 16):
                zero_v[i, pl.ds(j * 16, 16)] = zeros16
        for r in range(_ROWS_PER_TILE):
            ones_v[r, pl.ds(0, 16)] = one_hot16
            for j in range(1, _PW // 16):
                ones_v[r, pl.ds(j * 16, 16)] = zeros16

        def body(i, carry):
            b = cid * _IMGS_PER_CORE + i
            pltpu.sync_copy(zero_v, acc_sh.at[pl.ds(sid * _OUT_ROWS, _OUT_ROWS)])
            pltpu.sync_copy(zero_v,
                            pacc_sh.at[pl.ds(sid * _OUT_ROWS, _OUT_ROWS)])
            plsc.subcore_barrier()
            pltpu.sync_copy(descs_hbm.at[b, pl.ds(sid * _ROWS_PER_TILE,
                                                  _ROWS_PER_TILE)], rows_v)
            pltpu.sync_copy(clusters_hbm.at[b, pl.ds(sid * _ROWS_PER_TILE,
                                                     _ROWS_PER_TILE)], idx_v)
            pltpu.async_copy(rows_v, acc_sh.at[idx_v], sem, add=True).wait()
            pltpu.async_copy(ones_v, pacc_sh.at[idx_v], sem, add=True).wait()
            plsc.subcore_barrier()
            pltpu.sync_copy(acc_sh.at[pl.ds(sid * _OUT_ROWS, _OUT_ROWS)],
                            out_hbm.at[b, pl.ds(sid * _OUT_ROWS, _OUT_ROWS)])
            pltpu.sync_copy(pacc_sh.at[pl.ds(sid * _OUT_ROWS, _OUT_ROWS)],
                            pops_hbm.at[b, pl.ds(sid * _OUT_ROWS, _OUT_ROWS)])
            return carry

        lax.fori_loop(0, _IMGS_PER_CORE, body, 0)

    return seg_sum(descs, clusters)


_NB = 16


def _normalize_kernel(cent_ref, dsums_ref, pops_ref, out_ref):
    cent = cent_ref[...]
    desc_sums = dsums_ref[...]
    pops = pops_ref[:, :, 0]
    r = cent[None] * pops[:, :, None] - desc_sums

    g = jax.lax.dot_general(
        r, r, (((1,), (1,)), ((0,), (0,))),
        preferred_element_type=jnp.float32,
        precision=jax.lax.Precision.HIGHEST)
    eye = (jax.lax.broadcasted_iota(jnp.int32, (_NB, D, D), 1)
           == jax.lax.broadcasted_iota(jnp.int32, (_NB, D, D), 2))
    tr = jnp.sum(jnp.where(eye, g, 0.0), axis=(1, 2))
    h = (g / tr[:, None, None]).astype(jnp.bfloat16)
    for _ in range(NUM_SQUARINGS):
        h2 = jax.lax.dot_general(
            h, h, (((2,), (1,)), ((0,), (0,))),
            preferred_element_type=jnp.float32)
        tr2 = jnp.sum(jnp.where(eye, h2, 0.0), axis=(1, 2))
        h = (h2 / tr2[:, None, None]).astype(jnp.bfloat16)
    y = jnp.sum(h.astype(jnp.float32), axis=2)
    z = jnp.sum(g * y[:, :, None], axis=1)
    lam = jnp.sum(y * z, axis=1) / jnp.sum(y * y, axis=1)
    out_ref[...] = r * jax.lax.rsqrt(lam)[:, None, None]


def _tc_normalize(centroids, desc_sums, pops):
    return pl.pallas_call(
        _normalize_kernel,
        grid=(B // _NB,),
        in_specs=[
            pl.BlockSpec((K, D), lambda b: (0, 0)),
            pl.BlockSpec((_NB, K, D), lambda b: (b, 0, 0)),
            pl.BlockSpec((_NB, K, _PW), lambda b: (b, 0, 0)),
        ],
        out_specs=pl.BlockSpec((_NB, K, D), lambda b: (b, 0, 0)),
        out_shape=jax.ShapeDtypeStruct((B, K, D), jnp.float32),
    )(centroids, desc_sums, pops)


@jax.jit
def kernel(descs, centroids_sum, populations):
    centroids = centroids_sum / populations[:, None]
    c2 = jnp.sum(centroids * centroids, axis=-1)[None, :]
    clusters = _tc_assign(centroids, c2, descs)
    desc_sums, pops = _sc_segment_sum(descs, clusters)
    return _tc_normalize(centroids, desc_sums, pops)


# speedup vs baseline: 1.3668x; 1.3668x over previous
import functools

import jax
import jax.numpy as jnp
from jax import lax
from jax.experimental import pallas as pl
from jax.experimental.pallas import tpu as pltpu
from jax.experimental.pallas import tpu_sc as plsc

B, N, D, K = 64, 1024, 128, 256
NUM_SQUARINGS = 7


_AB = 1


def _assign_kernel(cent_ref, c2_ref, descs_ref, clusters_ref, pops_ref):
    cent = cent_ref[...]
    c2 = c2_ref[...]
    d = descs_ref[...].reshape(_AB * N, D)
    scores = jax.lax.dot_general(
        d, cent, (((1,), (1,)), ((), ())),
        preferred_element_type=jnp.float32,
        precision=jax.lax.Precision.HIGHEST)
    dist = c2 - 2.0 * scores
    clusters = jnp.argmin(dist, axis=-1)
    k_iota = jax.lax.broadcasted_iota(jnp.int32, (_AB * N, K), 1)
    onehot = (clusters[:, None] == k_iota).astype(jnp.float32)
    clusters_ref[...] = clusters.reshape(_AB, 1, N)
    pops_ref[...] = jnp.sum(onehot.reshape(_AB, N, K), axis=1)[:, None, :]


def _tc_assign(centroids, c2, descs):
    clusters, pops = pl.pallas_call(
        _assign_kernel,
        grid=(B // _AB,),
        in_specs=[
            pl.BlockSpec((K, D), lambda b: (0, 0)),
            pl.BlockSpec((1, K), lambda b: (0, 0)),
            pl.BlockSpec((_AB, N, D), lambda b: (b, 0, 0)),
        ],
        out_specs=[
            pl.BlockSpec((_AB, 1, N), lambda b: (b, 0, 0)),
            pl.BlockSpec((_AB, 1, K), lambda b: (b, 0, 0)),
        ],
        out_shape=[
            jax.ShapeDtypeStruct((B, 1, N), jnp.int32),
            jax.ShapeDtypeStruct((B, 1, K), jnp.float32),
        ],
    )(centroids, c2, descs)
    return clusters.reshape(B, N), pops.reshape(B, K)


_SC_INFO = plsc.get_sparse_core_info()
_NC = _SC_INFO.num_cores
_NS = _SC_INFO.num_subcores
_IMGS_PER_CORE = B // _NC
_ROWS_PER_TILE = N // _NS
_OUT_ROWS = K // _NS


def _sc_segment_sum(descs, clusters):
    mesh = plsc.VectorSubcoreMesh(core_axis_name="c", subcore_axis_name="s")

    @functools.partial(
        pl.kernel,
        out_type=jax.ShapeDtypeStruct((B, K, D), jnp.float32),
        mesh=mesh,
        scratch_types=[
            pltpu.VMEM((2, _ROWS_PER_TILE, D), jnp.float32),
            pltpu.VMEM((2, _ROWS_PER_TILE), jnp.int32),
            pltpu.VMEM((_OUT_ROWS, D), jnp.float32),
            pltpu.VMEM_SHARED((K, D), jnp.float32),
            pltpu.SemaphoreType.DMA,
            pltpu.SemaphoreType.DMA,
        ],
    )
    def seg_sum(descs_hbm, clusters_hbm, out_hbm, rows_v, idx_v, zero_v,
                acc_sh, sem, stage_sem):
        cid = lax.axis_index("c")
        sid = lax.axis_index("s")
        zeros16 = jnp.zeros((16,), jnp.float32)
        for i in range(_OUT_ROWS):
            for j in range(D // 16):
                zero_v[i, pl.ds(j * 16, 16)] = zeros16

        row_lo = sid * _ROWS_PER_TILE
        acc_rows = acc_sh.at[pl.ds(sid * _OUT_ROWS, _OUT_ROWS)]

        def stage(i, buf):
            b = cid * _IMGS_PER_CORE + i
            return (pltpu.async_copy(
                        descs_hbm.at[b, pl.ds(row_lo, _ROWS_PER_TILE)],
                        rows_v.at[buf], stage_sem),
                    pltpu.async_copy(
                        clusters_hbm.at[b, pl.ds(row_lo, _ROWS_PER_TILE)],
                        idx_v.at[buf], stage_sem))

        pending = stage(0, 0)
        for i in range(_IMGS_PER_CORE):
            b = cid * _IMGS_PER_CORE + i
            buf = i % 2
            pltpu.sync_copy(zero_v, acc_rows)
            for c in pending:
                c.wait()
            if i + 1 < _IMGS_PER_CORE:
                pending = stage(i + 1, 1 - buf)
            plsc.subcore_barrier()
            pltpu.async_copy(rows_v.at[buf], acc_sh.at[idx_v.at[buf]],
                             sem, add=True).wait()
            plsc.subcore_barrier()
            pltpu.sync_copy(acc_rows,
                            out_hbm.at[b, pl.ds(sid * _OUT_ROWS, _OUT_ROWS)])

    return seg_sum(descs, clusters)


_NB = 16


def _normalize_kernel(cent_ref, dsums_ref, pops_ref, out_ref):
    cent = cent_ref[...]
    desc_sums = dsums_ref[...]
    pops = pops_ref[:, 0, :]
    r = cent[None] * pops[:, :, None] - desc_sums

    g = jax.lax.dot_general(
        r, r, (((1,), (1,)), ((0,), (0,))),
        preferred_element_type=jnp.float32,
        precision=jax.lax.Precision.HIGHEST)
    eye = (jax.lax.broadcasted_iota(jnp.int32, (_NB, D, D), 1)
           == jax.lax.broadcasted_iota(jnp.int32, (_NB, D, D), 2))
    tr = jnp.sum(jnp.where(eye, g, 0.0), axis=(1, 2))
    h = (g / tr[:, None, None]).astype(jnp.bfloat16)
    for _ in range(NUM_SQUARINGS):
        h2 = jax.lax.dot_general(
            h, h, (((2,), (1,)), ((0,), (0,))),
            preferred_element_type=jnp.float32)
        tr2 = jnp.sum(jnp.where(eye, h2, 0.0), axis=(1, 2))
        h = (h2 / tr2[:, None, None]).astype(jnp.bfloat16)
    y = jnp.sum(h.astype(jnp.float32), axis=2)
    z = jnp.sum(g * y[:, :, None], axis=1)
    lam = jnp.sum(y * z, axis=1) / jnp.sum(y * y, axis=1)
    out_ref[...] = r * jax.lax.rsqrt(lam)[:, None, None]


def _tc_normalize(centroids, desc_sums, pops):
    return pl.pallas_call(
        _normalize_kernel,
        grid=(B // _NB,),
        in_specs=[
            pl.BlockSpec((K, D), lambda b: (0, 0)),
            pl.BlockSpec((_NB, K, D), lambda b: (b, 0, 0)),
            pl.BlockSpec((_NB, 1, K), lambda b: (b, 0, 0)),
        ],
        out_specs=pl.BlockSpec((_NB, K, D), lambda b: (b, 0, 0)),
        out_shape=jax.ShapeDtypeStruct((B, K, D), jnp.float32),
    )(centroids, desc_sums, pops)


@jax.jit
def kernel(descs, centroids_sum, populations):
    centroids = centroids_sum / populations[:, None]
    c2 = jnp.sum(centroids * centroids, axis=-1)[None, :]
    clusters, pops = _tc_assign(centroids, c2, descs)
    desc_sums = _sc_segment_sum(descs, clusters)
    return _tc_normalize(centroids, desc_sums, pops.reshape(B, 1, K))
